# baseline (speedup 1.0000x reference)
import functools

import jax
import jax.numpy as jnp
from jax.experimental import pallas as pl
from jax.experimental.pallas import tpu as pltpu

_TM = 256
_RES = 9


def _acc_layer(num_adjs, get_slab, h, w_ref, b_ref):
    acc = None
    for a in range(num_adjs):
        agg = jnp.dot(get_slab(a), h,
                      preferred_element_type=jnp.float32)
        part = jnp.dot(agg.astype(jnp.bfloat16), w_ref[a],
                       preferred_element_type=jnp.float32)
        acc = part if acc is None else acc + part
    return jnp.maximum(acc + b_ref[...], 0.0)


def _fused_kernel(num_adjs, num_tiles, num_res,
                  x_ref, w1_ref, b1_ref, w2_ref, b2_ref, adj_ref,
                  o_ref, adj_sc, x1_sc):
    s = pl.program_id(0)
    tm = o_ref.shape[0]
    tails = num_tiles - num_res

    @pl.when(s < num_tiles)
    def _phase1():
        j = s
        slot = jnp.minimum(j, num_res)
        for a in range(num_adjs):
            adj_sc[slot, a] = adj_ref[a].astype(jnp.bfloat16)
        x1t = _acc_layer(num_adjs, lambda a: adj_sc[slot, a],
                         x_ref[...], w1_ref, b1_ref)
        x1_sc[pl.ds(j * tm, tm), :] = x1t.astype(jnp.bfloat16)

    @pl.when(s >= num_tiles)
    def _phase2():
        k = s - num_tiles
        is_tail = k % 5 < 2

        @pl.when(is_tail)
        def _stage():
            for a in range(num_adjs):
                adj_sc[num_res, a] = adj_ref[a].astype(jnp.bfloat16)

        slot = jnp.where(is_tail, num_res, 3 * (k // 5) + k % 5 - 2)
        out = _acc_layer(num_adjs, lambda a: adj_sc[slot, a],
                         x1_sc[...], w2_ref, b2_ref)
        o_ref[...] = out.astype(o_ref.dtype)


def _fused_call(x_bf, adjs, w1_bf, b1_v, w2_bf, b2_v, out_dtype):
    A, N, _ = adjs.shape
    F = x_bf.shape[1]
    H = w1_bf.shape[2]
    T = N // _TM
    R = _RES
    tails = T - R

    def _tile2(k):
        return jnp.where(k % 5 < 2, T - 1 - (2 * (k // 5) + k % 5),
                         3 * (k // 5) + k % 5 - 2)

    def adj_map(s):
        k = s - T
        blk2 = T - 1 - (2 * (k // 5) + jnp.minimum(k % 5, 1))
        return (0, jnp.where(s < T, s, blk2), 0)

    def out_map(s):
        return (jnp.where(s < T, T - 1, _tile2(s - T)), 0)

    const = lambda s: (0, 0)
    const3 = lambda s: (0, 0, 0)

    kern = functools.partial(_fused_kernel, A, T, R)
    return pl.pallas_call(
        kern,
        out_shape=jax.ShapeDtypeStruct((N, H), out_dtype),
        grid=(2 * T,),
        in_specs=[
            pl.BlockSpec((N, F), const),
            pl.BlockSpec((A, F, H), const3),
            pl.BlockSpec((1, H), const),
            pl.BlockSpec((A, H, H), const3),
            pl.BlockSpec((1, H), const),
            pl.BlockSpec((A, _TM, N), adj_map),
        ],
        out_specs=pl.BlockSpec((_TM, H), out_map),
        scratch_shapes=[
            pltpu.VMEM((R + 1, A, _TM, N), jnp.bfloat16),
            pltpu.VMEM((N, H), jnp.bfloat16),
        ],
        compiler_params=pltpu.CompilerParams(
            dimension_semantics=("arbitrary",),
            vmem_limit_bytes=int(0.999 * 64 * 1024 * 1024)),
    )(x_bf, w1_bf, b1_v, w2_bf, b2_v, adjs)


def _layer_kernel(num_adjs, adj_ref, h_ref, w_ref, b_ref, o_ref):
    h = h_ref[...].astype(jnp.bfloat16)
    out = _acc_layer(num_adjs, lambda a: adj_ref[a].astype(jnp.bfloat16),
                     h, w_ref, b_ref)
    o_ref[...] = out.astype(o_ref.dtype)


def _layer(adjs, h, w_bf, b_v, row_tile, out_dtype):
    A, N, _ = adjs.shape
    F = h.shape[1]
    H = w_bf.shape[2]
    kern = functools.partial(_layer_kernel, A)
    return pl.pallas_call(
        kern,
        out_shape=jax.ShapeDtypeStruct((N, H), out_dtype),
        grid=(N // row_tile,),
        in_specs=[
            pl.BlockSpec((A, row_tile, N), lambda i: (0, i, 0)),
            pl.BlockSpec((N, F), lambda i: (0, 0)),
            pl.BlockSpec((A, F, H), lambda i: (0, 0, 0)),
            pl.BlockSpec((1, H), lambda i: (0, 0)),
        ],
        out_specs=pl.BlockSpec((row_tile, H), lambda i: (i, 0)),
        compiler_params=pltpu.CompilerParams(
            dimension_semantics=("parallel",),
            vmem_limit_bytes=int(0.9 * 64 * 1024 * 1024)),
    )(adjs, h, w_bf, b_v)


def kernel(x, adjs, w1, b1, w2, b2):
    N, F = x.shape
    H = w1.shape[2]
    out_dtype = x.dtype

    x_bf = x.astype(jnp.bfloat16)
    w1_bf = w1.astype(jnp.bfloat16)
    w2_bf = w2.astype(jnp.bfloat16)
    b1_v = b1.astype(jnp.float32).reshape(1, H)
    b2_v = b2.astype(jnp.float32).reshape(1, H)

    if N % _TM == 0 and N // _TM > _RES:
        return _fused_call(x_bf, adjs, w1_bf, b1_v, w2_bf, b2_v, out_dtype)

    row_tile = 256 if N % 256 == 0 else 8
    x1 = _layer(adjs, x_bf, w1_bf, b1_v, row_tile, jnp.bfloat16)
    return _layer(adjs, x1, w2_bf, b2_v, row_tile, out_dtype)


# speedup vs baseline: 1.0434x; 1.0434x over previous
import functools

import jax
import jax.numpy as jnp
from jax.experimental import pallas as pl
from jax.experimental.pallas import tpu as pltpu

_TM = 256
_RES = 9


def _acc_layer(num_adjs, get_slab, h, w_ref, b_ref):
    acc = None
    for a in range(num_adjs):
        agg = jnp.dot(get_slab(a), h,
                      preferred_element_type=jnp.float32)
        part = jnp.dot(agg.astype(jnp.bfloat16), w_ref[a],
                       preferred_element_type=jnp.float32)
        acc = part if acc is None else acc + part
    return jnp.maximum(acc + b_ref[...], 0.0)


def _fused_kernel(num_adjs, num_tiles, num_res,
                  x_ref, w1_ref, b1_ref, w2_ref, b2_ref, adj_ref,
                  o_ref, adj_sc, x1_sc):
    s = pl.program_id(0)
    tm = o_ref.shape[0]
    tails = num_tiles - num_res

    @pl.when(s < num_tiles)
    def _phase1():
        j = s
        slot = jnp.minimum(j, num_res)
        for a in range(num_adjs):
            adj_sc[slot, a] = adj_ref[a].astype(jnp.bfloat16)
        x1t = _acc_layer(num_adjs, lambda a: adj_sc[slot, a],
                         x_ref[...], w1_ref, b1_ref)
        x1_sc[pl.ds(j * tm, tm), :] = x1t.astype(jnp.bfloat16)

    @pl.when(s >= num_tiles)
    def _phase2():
        k = s - num_tiles
        is_tail = k < tails

        @pl.when(is_tail)
        def _stage():
            for a in range(num_adjs):
                adj_sc[num_res, a] = adj_ref[a].astype(jnp.bfloat16)

        slot = jnp.where(is_tail, num_res, k - tails)
        out = _acc_layer(num_adjs, lambda a: adj_sc[slot, a],
                         x1_sc[...], w2_ref, b2_ref)
        o_ref[...] = out.astype(o_ref.dtype)


def _fused_call(x_bf, adjs, w1_bf, b1_v, w2_bf, b2_v, out_dtype):
    A, N, _ = adjs.shape
    F = x_bf.shape[1]
    H = w1_bf.shape[2]
    T = N // _TM
    R = _RES
    tails = T - R

    def _tile2(k):
        return jnp.where(k < tails, T - 1 - k, k - tails)

    def adj_map(s):
        k = s - T
        blk2 = T - 1 - jnp.minimum(k, tails - 1)
        return (0, jnp.where(s < T, s, blk2), 0)

    def out_map(s):
        return (jnp.where(s < T, T - 1, _tile2(s - T)), 0)

    const = lambda s: (0, 0)
    const3 = lambda s: (0, 0, 0)

    kern = functools.partial(_fused_kernel, A, T, R)
    return pl.pallas_call(
        kern,
        out_shape=jax.ShapeDtypeStruct((N, H), out_dtype),
        grid=(2 * T,),
        in_specs=[
            pl.BlockSpec((N, F), const),
            pl.BlockSpec((A, F, H), const3),
            pl.BlockSpec((1, H), const),
            pl.BlockSpec((A, H, H), const3),
            pl.BlockSpec((1, H), const),
            pl.BlockSpec((A, _TM, N), adj_map),
        ],
        out_specs=pl.BlockSpec((_TM, H), out_map),
        scratch_shapes=[
            pltpu.VMEM((R + 1, A, _TM, N), jnp.bfloat16),
            pltpu.VMEM((N, H), jnp.bfloat16),
        ],
        compiler_params=pltpu.CompilerParams(
            dimension_semantics=("arbitrary",),
            vmem_limit_bytes=int(0.999 * 64 * 1024 * 1024)),
    )(x_bf, w1_bf, b1_v, w2_bf, b2_v, adjs)


def _layer_kernel(num_adjs, adj_ref, h_ref, w_ref, b_ref, o_ref):
    h = h_ref[...].astype(jnp.bfloat16)
    out = _acc_layer(num_adjs, lambda a: adj_ref[a].astype(jnp.bfloat16),
                     h, w_ref, b_ref)
    o_ref[...] = out.astype(o_ref.dtype)


def _layer(adjs, h, w_bf, b_v, row_tile, out_dtype):
    A, N, _ = adjs.shape
    F = h.shape[1]
    H = w_bf.shape[2]
    kern = functools.partial(_layer_kernel, A)
    return pl.pallas_call(
        kern,
        out_shape=jax.ShapeDtypeStruct((N, H), out_dtype),
        grid=(N // row_tile,),
        in_specs=[
            pl.BlockSpec((A, row_tile, N), lambda i: (0, i, 0)),
            pl.BlockSpec((N, F), lambda i: (0, 0)),
            pl.BlockSpec((A, F, H), lambda i: (0, 0, 0)),
            pl.BlockSpec((1, H), lambda i: (0, 0)),
        ],
        out_specs=pl.BlockSpec((row_tile, H), lambda i: (i, 0)),
        compiler_params=pltpu.CompilerParams(
            dimension_semantics=("parallel",),
            vmem_limit_bytes=int(0.9 * 64 * 1024 * 1024)),
    )(adjs, h, w_bf, b_v)


def kernel(x, adjs, w1, b1, w2, b2):
    N, F = x.shape
    H = w1.shape[2]
    out_dtype = x.dtype

    x_bf = x.astype(jnp.bfloat16)
    w1_bf = w1.astype(jnp.bfloat16)
    w2_bf = w2.astype(jnp.bfloat16)
    b1_v = b1.astype(jnp.float32).reshape(1, H)
    b2_v = b2.astype(jnp.float32).reshape(1, H)

    if N % _TM == 0 and N // _TM > _RES:
        return _fused_call(x_bf, adjs, w1_bf, b1_v, w2_bf, b2_v, out_dtype)

    row_tile = 256 if N % 256 == 0 else 8
    x1 = _layer(adjs, x_bf, w1_bf, b1_v, row_tile, jnp.bfloat16)
    return _layer(adjs, x1, w2_bf, b2_v, row_tile, out_dtype)
